# baseline (speedup 1.0000x reference)
import functools

import jax
import jax.numpy as jnp
from jax import lax
from jax.experimental import pallas as pl
from jax.experimental.pallas import tpu as pltpu
from jax.experimental.pallas import tpu_sc as plsc

D_MODEL = 2048
N_TOK = 16384
NC, NS, L = 2, 16, 16
NW = NC * NS
ROWS_PER_W = N_TOK // NW
CHUNK = 32
N_CHUNKS = ROWS_PER_W // CHUNK
BUF = CHUNK * D_MODEL


def _sc_body(zeros_hbm, idx_hbm, out_hbm, buf, idx_v):
    wid = lax.axis_index("s") * NC + lax.axis_index("c")
    base = wid * ROWS_PER_W
    pltpu.sync_copy(zeros_hbm, buf)
    pltpu.sync_copy(idx_hbm.at[pl.ds(base, ROWS_PER_W)], idx_v)
    rowoff = lax.iota(jnp.int32, L) * D_MODEL
    one = jnp.full((L,), 1.0, jnp.float32)
    zero = jnp.zeros((L,), jnp.float32)

    def chunk_step(c, _):
        for g in range(CHUNK // L):
            cols = idx_v[pl.ds(c * CHUNK + g * L, L)]
            plsc.store_scatter(buf, [rowoff + (g * L * D_MODEL) + cols], one)
        pltpu.sync_copy(buf, out_hbm.at[pl.ds((base + c * CHUNK) * D_MODEL, BUF)])
        for g in range(CHUNK // L):
            cols = idx_v[pl.ds(c * CHUNK + g * L, L)]
            plsc.store_scatter(buf, [rowoff + (g * L * D_MODEL) + cols], zero)
        return _

    lax.fori_loop(0, N_CHUNKS, chunk_step, None)


def kernel(x):
    b, s, _ = x.shape
    idx = x.reshape(N_TOK)
    zeros = jnp.zeros((BUF,), jnp.float32)
    mesh = plsc.VectorSubcoreMesh(core_axis_name="c", subcore_axis_name="s")
    k = functools.partial(
        pl.kernel,
        mesh=mesh,
        out_type=jax.ShapeDtypeStruct((N_TOK * D_MODEL,), jnp.float32),
        scratch_types=[
            pltpu.VMEM((BUF,), jnp.float32),
            pltpu.VMEM((ROWS_PER_W,), jnp.int32),
        ],
        compiler_params=pltpu.CompilerParams(needs_layout_passes=False),
    )(_sc_body)
    out = k(zeros, idx)
    return (out.reshape(b, s, D_MODEL),)
